# baseline (speedup 1.0000x reference)
import functools

import jax
import jax.numpy as jnp
from jax import lax
from jax.experimental import pallas as pl
from jax.experimental.pallas import tpu as pltpu, tpu_sc as plsc

L = 16
NW = 32
C = 128


def _softmax_embed(x_flat, weight):
    n, d = x_flat.shape[0], weight.shape[1]
    rows_per_w = n // NW
    n_chunks = rows_per_w // C
    mesh = plsc.VectorSubcoreMesh(core_axis_name="c", subcore_axis_name="s")

    @functools.partial(
        pl.kernel,
        out_type=jax.ShapeDtypeStruct((n, d), jnp.float32),
        mesh=mesh,
        compiler_params=pltpu.CompilerParams(
            needs_layout_passes=False, use_tc_tiling_on_sc=False),
        scratch_types=[
            pltpu.VMEM((C,), jnp.int32),
            pltpu.VMEM((C, d), jnp.float32),
            pltpu.VMEM((C, d), jnp.float32),
            pltpu.SemaphoreType.DMA,
        ],
    )
    def k(x_hbm, w_hbm, out_hbm, idx_v, rows_v, outb_v, sem):
        wid = lax.axis_index("s") * 2 + lax.axis_index("c")
        base = wid * rows_per_w
        lanes = lax.iota(jnp.int32, L)

        def chunk_body(g, carry):
            start = base + g * C
            pltpu.sync_copy(x_hbm.at[pl.ds(start, C)], idx_v)
            pltpu.async_copy(w_hbm.at[idx_v], rows_v, sem).wait()

            def group_body(t, carry):
                r0 = t * L
                rows16 = r0 + lanes
                idx16 = idx_v[pl.ds(r0, L)]
                s = jnp.zeros((L,), jnp.float32)
                for j in range(d):
                    col = jnp.full((L,), j, jnp.int32)
                    cj = plsc.load_gather(rows_v, [rows16, col])
                    s = s + jnp.exp(cj)
                r = jnp.where(idx16 == 0, 0.0, 1.0 / s)
                for j in range(d):
                    col = jnp.full((L,), j, jnp.int32)
                    cj = plsc.load_gather(rows_v, [rows16, col])
                    plsc.store_scatter(outb_v, [rows16, col], jnp.exp(cj) * r)
                return carry

            lax.fori_loop(0, C // L, group_body, 0)
            pltpu.sync_copy(outb_v, out_hbm.at[pl.ds(start, C)])
            return carry

        lax.fori_loop(0, n_chunks, chunk_body, 0)

    return k(x_flat, weight)


def kernel(x, weight):
    b, h = x.shape
    out = _softmax_embed(x.reshape(b * h), weight)
    return out.reshape(b, h, weight.shape[1])


# speedup vs baseline: 1.0767x; 1.0767x over previous
import functools

import jax
import jax.numpy as jnp
from jax import lax
from jax.experimental import pallas as pl
from jax.experimental.pallas import tpu as pltpu, tpu_sc as plsc

L = 16
NW = 32
C = 128


def _softmax_embed(x_flat, weight):
    n, d = x_flat.shape[0], weight.shape[1]
    rows_per_w = n // NW
    n_chunks = rows_per_w // C
    mesh = plsc.VectorSubcoreMesh(core_axis_name="c", subcore_axis_name="s")

    @functools.partial(
        pl.kernel,
        out_type=jax.ShapeDtypeStruct((n, d), jnp.float32),
        mesh=mesh,
        compiler_params=pltpu.CompilerParams(
            needs_layout_passes=False, use_tc_tiling_on_sc=False),
        scratch_types=[
            pltpu.VMEM((rows_per_w,), jnp.int32),
            pltpu.VMEM((2, C, d), jnp.float32),
            pltpu.VMEM((2, C, d), jnp.float32),
            pltpu.SemaphoreType.DMA((2,)),
            pltpu.SemaphoreType.DMA((2,)),
        ],
    )
    def k(x_hbm, w_hbm, out_hbm, idx_v, rows_v, outb_v, gsem, osem):
        wid = lax.axis_index("s") * 2 + lax.axis_index("c")
        base = wid * rows_per_w
        lanes = lax.iota(jnp.int32, L)
        pltpu.sync_copy(x_hbm.at[pl.ds(base, rows_per_w)], idx_v)

        def gather(g, b):
            return pltpu.make_async_copy(
                w_hbm.at[idx_v.at[pl.ds(g * C, C)]], rows_v.at[b], gsem.at[b])

        def scatter(g, b):
            return pltpu.make_async_copy(
                outb_v.at[b], out_hbm.at[pl.ds(base + g * C, C)], osem.at[b])

        def compute(g, b):
            rows_b, outb_b = rows_v.at[b], outb_v.at[b]

            def group_body(t, carry):
                rows16 = t * L + lanes
                idx16 = idx_v[pl.ds(g * C + t * L, L)]
                part = [jnp.zeros((L,), jnp.float32) for _ in range(4)]
                for j in range(d):
                    col = jnp.full((L,), j, jnp.int32)
                    cj = plsc.load_gather(rows_b, [rows16, col])
                    part[j % 4] = part[j % 4] + jnp.exp(cj)
                s = (part[0] + part[1]) + (part[2] + part[3])
                r = jnp.where(idx16 == 0, 0.0, 1.0 / s)
                for j in range(d):
                    col = jnp.full((L,), j, jnp.int32)
                    cj = plsc.load_gather(rows_b, [rows16, col])
                    plsc.store_scatter(outb_b, [rows16, col], jnp.exp(cj) * r)
                return carry

            lax.fori_loop(0, C // L, group_body, 0)

        gather(0, 0).start()
        gather(1, 1).start()

        def chunk_body(g, carry):
            b = lax.rem(g, 2)

            def one(b):
                gather(g, b).wait()

                @pl.when(g >= 2)
                def _():
                    scatter(g, b).wait()

                compute(g, b)
                scatter(g, b).start()

                @pl.when(g < n_chunks - 2)
                def _():
                    gather(g + 2, b).start()

            @pl.when(b == 0)
            def _():
                one(0)

            @pl.when(b == 1)
            def _():
                one(1)

            return carry

        lax.fori_loop(0, n_chunks, chunk_body, 0)
        scatter(n_chunks - 2, 0).wait()
        scatter(n_chunks - 1, 1).wait()

    return k(x_flat, weight)


def kernel(x, weight):
    b, h = x.shape
    out = _softmax_embed(x.reshape(b * h), weight)
    return out.reshape(b, h, weight.shape[1])


# speedup vs baseline: 2.0038x; 1.8610x over previous
import functools

import jax
import jax.numpy as jnp
from jax import lax
from jax.experimental import pallas as pl
from jax.experimental.pallas import tpu as pltpu, tpu_sc as plsc

L = 16
NW = 32
C = 128


def _softmax_embed(x_flat, weight):
    n, d = x_flat.shape[0], weight.shape[1]
    rows_per_w = n // NW
    n_chunks = rows_per_w // C
    mesh = plsc.VectorSubcoreMesh(core_axis_name="c", subcore_axis_name="s")

    @functools.partial(
        pl.kernel,
        out_type=jax.ShapeDtypeStruct((n, d), jnp.float32),
        mesh=mesh,
        compiler_params=pltpu.CompilerParams(
            needs_layout_passes=False, use_tc_tiling_on_sc=False),
        scratch_types=[
            pltpu.VMEM((rows_per_w,), jnp.int32),
            pltpu.VMEM((2, C, d), jnp.float32),
            pltpu.VMEM((2, C, d), jnp.float32),
            pltpu.SemaphoreType.DMA((2,)),
            pltpu.SemaphoreType.DMA((2,)),
        ],
    )
    def k(x_hbm, w_hbm, out_hbm, idx_v, rows_v, outb_v, gsem, osem):
        wid = lax.axis_index("s") * 2 + lax.axis_index("c")
        base = wid * rows_per_w
        lanes = lax.iota(jnp.int32, L)
        pltpu.sync_copy(x_hbm.at[pl.ds(base, rows_per_w)], idx_v)

        def gather(g, b):
            return pltpu.make_async_copy(
                w_hbm.at[idx_v.at[pl.ds(g * C, C)]], rows_v.at[b], gsem.at[b])

        def scatter(g, b):
            return pltpu.make_async_copy(
                outb_v.at[b], out_hbm.at[pl.ds(base + g * C, C)], osem.at[b])

        def compute(g, b):
            rows_b, outb_b = rows_v.at[b], outb_v.at[b]

            def group_body(t, carry):
                r0 = t * L
                rows16 = r0 + lanes
                idx16 = idx_v[pl.ds(g * C + r0, L)]
                part = [jnp.zeros((L,), jnp.float32) for _ in range(4)]
                for j in range(d):
                    col = (lanes + j) & (d - 1)
                    cj = plsc.load_gather(rows_b, [rows16, col])
                    part[j % 4] = part[j % 4] + jnp.exp(cj)
                s = (part[0] + part[1]) + (part[2] + part[3])
                r = jnp.where(idx16 == 0, 0.0, 1.0 / s)
                for i in range(L):
                    ri = jnp.broadcast_to(r[i], (L,))
                    for c0 in range(0, d, L):
                        cj = rows_b[r0 + i, pl.ds(c0, L)]
                        outb_b[r0 + i, pl.ds(c0, L)] = jnp.exp(cj) * ri
                return carry

            lax.fori_loop(0, C // L, group_body, 0)

        gather(0, 0).start()
        gather(1, 1).start()

        def chunk_body(g, carry):
            b = lax.rem(g, 2)

            def one(b):
                gather(g, b).wait()

                @pl.when(g >= 2)
                def _():
                    scatter(g, b).wait()

                compute(g, b)
                scatter(g, b).start()

                @pl.when(g < n_chunks - 2)
                def _():
                    gather(g + 2, b).start()

            @pl.when(b == 0)
            def _():
                one(0)

            @pl.when(b == 1)
            def _():
                one(1)

            return carry

        lax.fori_loop(0, n_chunks, chunk_body, 0)
        scatter(n_chunks - 2, 0).wait()
        scatter(n_chunks - 1, 1).wait()

    return k(x_flat, weight)


def kernel(x, weight):
    b, h = x.shape
    out = _softmax_embed(x.reshape(b * h), weight)
    return out.reshape(b, h, weight.shape[1])


# speedup vs baseline: 2.9910x; 1.4927x over previous
import functools

import jax
import jax.numpy as jnp
from jax import lax
from jax.experimental import pallas as pl
from jax.experimental.pallas import tpu as pltpu, tpu_sc as plsc

L = 16
NW = 32
C = 128


def _softmax_embed(x_flat, bsz, hist, weight):
    n, d = bsz * hist, weight.shape[1]
    rows_per_w = n // NW
    n_chunks = rows_per_w // C
    mesh = plsc.VectorSubcoreMesh(core_axis_name="c", subcore_axis_name="s")

    @functools.partial(
        pl.kernel,
        out_type=jax.ShapeDtypeStruct((n, d), jnp.float32),
        mesh=mesh,
        compiler_params=pltpu.CompilerParams(
            needs_layout_passes=False, use_tc_tiling_on_sc=False),
        scratch_types=[
            pltpu.VMEM((rows_per_w,), jnp.int32),
            pltpu.VMEM((2, C, d), jnp.float32),
            pltpu.VMEM((2, C, d), jnp.float32),
            pltpu.SemaphoreType.DMA((2,)),
            pltpu.SemaphoreType.DMA((2,)),
        ],
    )
    def k(x_hbm, w_hbm, out_hbm, idx_v, rows_v, outb_v, gsem, osem):
        wid = lax.axis_index("s") * 2 + lax.axis_index("c")
        base = wid * rows_per_w
        lanes = lax.iota(jnp.int32, L)
        pltpu.sync_copy(x_hbm.at[pl.ds(base, rows_per_w)], idx_v)

        def gather(g, b):
            return pltpu.make_async_copy(
                w_hbm.at[idx_v.at[pl.ds(g * C, C)]], rows_v.at[b], gsem.at[b])

        def scatter(g, b):
            return pltpu.make_async_copy(
                outb_v.at[b], out_hbm.at[pl.ds(base + g * C, C)], osem.at[b])

        def compute(g, b):
            rows_b, outb_b = rows_v.at[b], outb_v.at[b]

            def group_body(t, carry):
                r0 = t * L
                rows16 = r0 + lanes
                idx16 = idx_v[pl.ds(g * C + r0, L)]
                part = [jnp.zeros((L,), jnp.float32) for _ in range(4)]
                for j in range(d):
                    col = (lanes + j) & (d - 1)
                    cj = plsc.load_gather(rows_b, [rows16, col])
                    part[j % 4] = part[j % 4] + jnp.exp(cj)
                s = (part[0] + part[1]) + (part[2] + part[3])
                r = jnp.where(idx16 == 0, 0.0, 1.0 / s)
                for i in range(L):
                    ri = jnp.broadcast_to(r[i], (L,))
                    cs = [rows_b[r0 + i, pl.ds(c0, L)]
                          for c0 in range(0, d, L)]
                    es = [jnp.exp(c) * ri for c in cs]
                    for blk, c0 in enumerate(range(0, d, L)):
                        outb_b[r0 + i, pl.ds(c0, L)] = es[blk]
                return carry

            lax.fori_loop(0, C // L, group_body, 0)

        gather(0, 0).start()
        gather(1, 1).start()

        def chunk_body(g, carry):
            b = lax.rem(g, 2)

            def one(b):
                gather(g, b).wait()

                @pl.when(g >= 2)
                def _():
                    scatter(g, b).wait()

                compute(g, b)
                scatter(g, b).start()

                @pl.when(g < n_chunks - 2)
                def _():
                    gather(g + 2, b).start()

            @pl.when(b == 0)
            def _():
                one(0)

            @pl.when(b == 1)
            def _():
                one(1)

            return carry

        lax.fori_loop(0, n_chunks, chunk_body, 0)
        scatter(n_chunks - 2, 0).wait()
        scatter(n_chunks - 1, 1).wait()

    return k(x_flat, weight)


def kernel(x, weight):
    b, h = x.shape
    out = _softmax_embed(x.reshape(b * h), b, h, weight)
    return out.reshape(b, h, weight.shape[1])


# speedup vs baseline: 4.5172x; 1.5102x over previous
import functools

import jax
import jax.numpy as jnp
from jax import lax
from jax.experimental import pallas as pl
from jax.experimental.pallas import tpu as pltpu, tpu_sc as plsc

L = 16
NW = 32
C = 128


def _softmax_embed(x_flat, bsz, hist, weight):
    d = weight.shape[1]
    b_per_w = bsz // NW
    rows_per_w = b_per_w * hist
    bg_per_w = b_per_w // C
    n_chunks = hist * bg_per_w
    mesh = plsc.VectorSubcoreMesh(core_axis_name="c", subcore_axis_name="s")

    @functools.partial(
        pl.kernel,
        out_type=jax.ShapeDtypeStruct((hist, d // 8, bsz // C, 8, C),
                                      jnp.float32),
        mesh=mesh,
        compiler_params=pltpu.CompilerParams(
            needs_layout_passes=False, use_tc_tiling_on_sc=False),
        scratch_types=[
            pltpu.VMEM((rows_per_w,), jnp.int32),
            pltpu.VMEM((2, C), jnp.int32),
            pltpu.VMEM((2, C, d), jnp.float32),
            pltpu.VMEM((2, d, C), jnp.float32),
            pltpu.SemaphoreType.DMA((2,)),
            pltpu.SemaphoreType.DMA((2,)),
        ],
    )
    def k(x_hbm, w_hbm, out_hbm, idx_v, idxc_v, rows_v, outb_v, gsem, osem):
        wid = lax.axis_index("s") * 2 + lax.axis_index("c")
        lanes = lax.iota(jnp.int32, L)
        pltpu.sync_copy(x_hbm.at[pl.ds(wid * rows_per_w, rows_per_w)], idx_v)

        def chunk_pos(g):
            return g // bg_per_w, wid * bg_per_w + lax.rem(g, bg_per_w)

        def stage_idx(g, b):
            h = g // bg_per_w
            lb0 = lax.rem(g, bg_per_w) * C
            for kk in range(C // L):
                src = (lb0 + kk * L + lanes) * hist + h
                v = plsc.load_gather(idx_v, [src])
                idxc_v[b, pl.ds(kk * L, L)] = v

        def gather(g, b):
            return pltpu.make_async_copy(
                w_hbm.at[idxc_v.at[b]], rows_v.at[b], gsem.at[b])

        def tile_write(g, b, cg):
            h, bg = chunk_pos(g)
            return pltpu.make_async_copy(
                outb_v.at[b, pl.ds(cg * 8, 8)], out_hbm.at[h, cg, bg],
                osem.at[b])

        def compute(g, b):
            rows_b, outb_b = rows_v.at[b], outb_v.at[b]

            def group_body(t, carry):
                r0 = t * L
                rows16 = r0 + lanes
                idx16 = idxc_v[b, pl.ds(r0, L)]
                part = [jnp.zeros((L,), jnp.float32) for _ in range(4)]
                for j in range(d):
                    col = (lanes + j) & (d - 1)
                    cj = plsc.load_gather(rows_b, [rows16, col])
                    part[j % 4] = part[j % 4] + jnp.exp(cj)
                s = (part[0] + part[1]) + (part[2] + part[3])
                r = jnp.where(idx16 == 0, 0.0, 1.0 / s)
                for c0 in range(0, d, L):
                    cols = [c0 + ((lanes + j) & (L - 1)) for j in range(L)]
                    vs = [plsc.load_gather(rows_b, [rows16, cj])
                          for cj in cols]
                    es = [jnp.exp(v) * r for v in vs]
                    for cj, e in zip(cols, es):
                        plsc.store_scatter(outb_b, [cj, rows16], e)
                return carry

            lax.fori_loop(0, C // L, group_body, 0)

        stage_idx(0, 0)
        gather(0, 0).start()
        stage_idx(1, 1)
        gather(1, 1).start()

        def chunk_body(g, carry):
            b = lax.rem(g, 2)

            def one(b):
                gather(g, b).wait()

                @pl.when(g >= 2)
                def _():
                    for cg in range(d // 8):
                        tile_write(g - 2, b, cg).wait()

                compute(g, b)
                for cg in range(d // 8):
                    tile_write(g, b, cg).start()

                @pl.when(g < n_chunks - 2)
                def _():
                    stage_idx(g + 2, b)
                    gather(g + 2, b).start()

            @pl.when(b == 0)
            def _():
                one(0)

            @pl.when(b == 1)
            def _():
                one(1)

            return carry

        lax.fori_loop(0, n_chunks, chunk_body, 0)
        for cg in range(d // 8):
            tile_write(n_chunks - 2, 0, cg).wait()
            tile_write(n_chunks - 1, 1, cg).wait()

    return k(x_flat, weight)


def kernel(x, weight):
    b, h = x.shape
    d = weight.shape[1]
    out5 = _softmax_embed(x.reshape(b * h), b, h, weight)
    return out5.transpose(0, 1, 3, 2, 4).reshape(h, d, b).transpose(2, 0, 1)


# speedup vs baseline: 4.8023x; 1.0631x over previous
import functools

import jax
import jax.numpy as jnp
from jax import lax
from jax.experimental import pallas as pl
from jax.experimental.pallas import tpu as pltpu, tpu_sc as plsc

L = 16
NW = 32
C = 128


def _softmax_embed(x_flat, bsz, hist, weight):
    d = weight.shape[1]
    b_per_w = bsz // NW
    rows_per_w = b_per_w * hist
    bg_per_w = b_per_w // C
    n_chunks = hist * bg_per_w
    mesh = plsc.VectorSubcoreMesh(core_axis_name="c", subcore_axis_name="s")

    @functools.partial(
        pl.kernel,
        out_type=jax.ShapeDtypeStruct((hist, d // 8, bsz // C, 8, C),
                                      jnp.float32),
        mesh=mesh,
        compiler_params=pltpu.CompilerParams(
            needs_layout_passes=False, use_tc_tiling_on_sc=False),
        scratch_types=[
            pltpu.VMEM((rows_per_w,), jnp.int32),
            pltpu.VMEM((2, C), jnp.int32),
            pltpu.VMEM((2, C, d), jnp.float32),
            pltpu.VMEM((2, d, C), jnp.float32),
            pltpu.SemaphoreType.DMA((2,)),
            pltpu.SemaphoreType.DMA((2,)),
        ],
    )
    def k(x_hbm, w_hbm, out_hbm, idx_v, idxc_v, rows_v, outb_v, gsem, osem):
        wid = lax.axis_index("s") * 2 + lax.axis_index("c")
        lanes = lax.iota(jnp.int32, L)
        pltpu.sync_copy(x_hbm.at[pl.ds(wid * rows_per_w, rows_per_w)], idx_v)

        def chunk_pos(g):
            return g // bg_per_w, wid * bg_per_w + lax.rem(g, bg_per_w)

        def stage_idx(g, b):
            h = g // bg_per_w
            lb0 = lax.rem(g, bg_per_w) * C
            for kk in range(C // L):
                src = (lb0 + kk * L + lanes) * hist + h
                v = plsc.load_gather(idx_v, [src])
                idxc_v[b, pl.ds(kk * L, L)] = v * 2

        def gather(g, b):
            return pltpu.make_async_copy(
                w_hbm.at[idxc_v.at[b]], rows_v.at[b], gsem.at[b])

        def tile_write(g, b, cg):
            h, bg = chunk_pos(g)
            return pltpu.make_async_copy(
                outb_v.at[b, pl.ds(cg * 8, 8)], out_hbm.at[h, cg, bg],
                osem.at[b])

        def compute(g, b):
            rows_b, outb_b = rows_v.at[b], outb_v.at[b]

            def group_body(t, carry):
                r0 = t * L
                rows16 = r0 + lanes
                idx16 = idxc_v[b, pl.ds(r0, L)]
                part = [jnp.zeros((L,), jnp.float32) for _ in range(4)]
                for j in range(d):
                    col = (lanes + j) & (d - 1)
                    cj = plsc.load_gather(rows_b, [rows16, col])
                    part[j % 4] = part[j % 4] + jnp.exp(cj)
                s = (part[0] + part[1]) + (part[2] + part[3])
                r = jnp.where(idx16 == 0, 0.0, 1.0 / s)
                for c0 in range(0, d, L):
                    cols = [c0 + ((lanes + j) & (L - 1)) for j in range(L)]
                    vs = [plsc.load_gather(rows_b, [rows16, cj])
                          for cj in cols]
                    es = [jnp.exp(v) * r for v in vs]
                    for cj, e in zip(cols, es):
                        plsc.store_scatter(outb_b, [cj, rows16], e)
                return carry

            lax.fori_loop(0, C // L, group_body, 0)

        stage_idx(0, 0)
        gather(0, 0).start()
        stage_idx(1, 1)
        gather(1, 1).start()

        def chunk_body(g, carry):
            b = lax.rem(g, 2)

            def one(b):
                gather(g, b).wait()

                @pl.when(g >= 2)
                def _():
                    for cg in range(d // 8):
                        tile_write(g - 2, b, cg).wait()

                compute(g, b)
                for cg in range(d // 8):
                    tile_write(g, b, cg).start()

                @pl.when(g < n_chunks - 2)
                def _():
                    stage_idx(g + 2, b)
                    gather(g + 2, b).start()

            @pl.when(b == 0)
            def _():
                one(0)

            @pl.when(b == 1)
            def _():
                one(1)

            return carry

        lax.fori_loop(0, n_chunks, chunk_body, 0)
        for cg in range(d // 8):
            tile_write(n_chunks - 2, 0, cg).wait()
            tile_write(n_chunks - 1, 1, cg).wait()

    return k(x_flat, weight)


def kernel(x, weight):
    b, h = x.shape
    n, d = weight.shape
    w2 = jnp.pad(weight, ((0, 0), (0, d))).reshape(2 * n, d)
    out5 = _softmax_embed(x.reshape(b * h), b, h, w2)
    return out5.transpose(0, 1, 3, 2, 4).reshape(h, d, b).transpose(2, 0, 1)


# speedup vs baseline: 6.3214x; 1.3163x over previous
import functools

import jax
import jax.numpy as jnp
from jax import lax
from jax.experimental import pallas as pl
from jax.experimental.pallas import tpu as pltpu, tpu_sc as plsc

L = 16
NW = 32
C = 128


def _transpose_table(wt, tail_w):
    d, n = wt.shape
    nb_full = n // C
    tail = n - nb_full * C
    mesh = plsc.VectorSubcoreMesh(core_axis_name="c", subcore_axis_name="s")

    @functools.partial(
        pl.kernel,
        out_type=jax.ShapeDtypeStruct((n // 2, 2 * d), jnp.float32),
        mesh=mesh,
        compiler_params=pltpu.CompilerParams(
            needs_layout_passes=False, use_tc_tiling_on_sc=True),
        scratch_types=[
            pltpu.VMEM((2, d, C), jnp.float32),
            pltpu.VMEM((2, C // 2, 2 * d), jnp.float32),
            pltpu.SemaphoreType.DMA((2,)),
            pltpu.SemaphoreType.DMA((2,)),
        ],
    )
    def k(wt_hbm, tail_hbm, out_hbm, in_v, out_v, isem, osem):
        wid = lax.axis_index("s") * 2 + lax.axis_index("c")
        lanes = lax.iota(jnp.int32, L)
        n_my = (nb_full - wid + NW - 1) // NW

        def load(t, b):
            nb = wid + t * NW
            return pltpu.make_async_copy(
                wt_hbm.at[:, pl.ds(nb * C, C)], in_v.at[b], isem.at[b])

        def store(t, b):
            nb = wid + t * NW
            return pltpu.make_async_copy(
                out_v.at[b], out_hbm.at[pl.ds(nb * (C // 2), C // 2)],
                osem.at[b])

        def transpose_block(b, width):
            in_b, out_b = in_v.at[b], out_v.at[b]

            def il_body(t, carry):
                il16 = t * L + lanes
                dst_r = il16 >> 1
                dst_base = (il16 & 1) * d
                for c0 in range(0, d, L):
                    cols = [c0 + ((lanes + j) & (L - 1)) for j in range(L)]
                    vs = [plsc.load_gather(in_b, [cj, il16]) for cj in cols]
                    for cj, v in zip(cols, vs):
                        plsc.store_scatter(out_b, [dst_r, dst_base + cj], v)
                return carry

            lax.fori_loop(0, width // L, il_body, 0)

        load(0, 0).start()
        load(1, 1).start()

        def block_body(t, carry):
            b = lax.rem(t, 2)

            def one(b):
                load(t, b).wait()

                @pl.when(t >= 2)
                def _():
                    store(t - 2, b).wait()

                transpose_block(b, C)
                store(t, b).start()

                @pl.when(t < n_my - 2)
                def _():
                    load(t + 2, b).start()

            @pl.when(b == 0)
            def _():
                one(0)

            @pl.when(b == 1)
            def _():
                one(1)

            return carry

        lax.fori_loop(0, n_my, block_body, 0)
        store(n_my - 2, 0).wait()
        store(n_my - 1, 1).wait()

        if tail:
            @pl.when(wid == 0)
            def _():
                tin = pltpu.make_async_copy(tail_hbm, in_v.at[0],
                                            isem.at[0])
                tin.start()
                tin.wait()
                transpose_block(0, C)
                tout = pltpu.make_async_copy(
                    out_v.at[0, pl.ds(0, tail // 2)],
                    out_hbm.at[pl.ds(nb_full * (C // 2), tail // 2)],
                    osem.at[0])
                tout.start()
                tout.wait()

    return k(wt, tail_w)


def _softmax_embed(x_flat, bsz, hist, weight):
    d = weight.shape[1] // 2
    b_per_w = bsz // NW
    rows_per_w = b_per_w * hist
    bg_per_w = b_per_w // C
    n_chunks = hist * bg_per_w
    mesh = plsc.VectorSubcoreMesh(core_axis_name="c", subcore_axis_name="s")

    @functools.partial(
        pl.kernel,
        out_type=jax.ShapeDtypeStruct((hist, d // 8, bsz // C, 8, C),
                                      jnp.float32),
        mesh=mesh,
        compiler_params=pltpu.CompilerParams(
            needs_layout_passes=False, use_tc_tiling_on_sc=True),
        scratch_types=[
            pltpu.VMEM((rows_per_w,), jnp.int32),
            pltpu.VMEM((2, C), jnp.int32),
            pltpu.VMEM((2, C), jnp.int32),
            pltpu.VMEM((2, C, 2 * d), jnp.float32),
            pltpu.VMEM((2, d, C), jnp.float32),
            pltpu.SemaphoreType.DMA((2,)),
            pltpu.SemaphoreType.DMA((2,)),
        ],
    )
    def k(x_hbm, w_hbm, out_hbm, idx_v, idxc_v, idxo_v, rows_v, outb_v,
          gsem, osem):
        wid = lax.axis_index("s") * 2 + lax.axis_index("c")
        lanes = lax.iota(jnp.int32, L)
        pltpu.sync_copy(x_hbm.at[pl.ds(wid * rows_per_w, rows_per_w)], idx_v)

        def chunk_pos(g):
            return g // bg_per_w, wid * bg_per_w + lax.rem(g, bg_per_w)

        def stage_idx(g, b):
            h = g // bg_per_w
            lb0 = lax.rem(g, bg_per_w) * C
            for kk in range(C // L):
                src = (lb0 + kk * L + lanes) * hist + h
                v = plsc.load_gather(idx_v, [src])
                idxc_v[b, pl.ds(kk * L, L)] = v >> 1
                idxo_v[b, pl.ds(kk * L, L)] = v

        def gather(g, b):
            return pltpu.make_async_copy(
                w_hbm.at[idxc_v.at[b]], rows_v.at[b], gsem.at[b])

        def tile_write(g, b, cg):
            h, bg = chunk_pos(g)
            return pltpu.make_async_copy(
                outb_v.at[b, pl.ds(cg * 8, 8)], out_hbm.at[h, cg, bg],
                osem.at[b])

        def compute(g, b):
            rows_b, outb_b = rows_v.at[b], outb_v.at[b]

            def group_body(t, carry):
                r0 = t * L
                rows16 = r0 + lanes
                idx16 = idxo_v[b, pl.ds(r0, L)]
                off16 = (idx16 & 1) * d
                part = [jnp.zeros((L,), jnp.float32) for _ in range(4)]
                for j in range(d):
                    col = (lanes + j) & (d - 1)
                    cj = plsc.load_gather(rows_b, [rows16, off16 + col])
                    part[j % 4] = part[j % 4] + jnp.exp(cj)
                s = (part[0] + part[1]) + (part[2] + part[3])
                r = jnp.where(idx16 == 0, 0.0, 1.0 / s)
                for c0 in range(0, d, L):
                    cols = [c0 + ((lanes + j) & (L - 1)) for j in range(L)]
                    vs = [plsc.load_gather(rows_b, [rows16, off16 + cj])
                          for cj in cols]
                    es = [jnp.exp(v) * r for v in vs]
                    for cj, e in zip(cols, es):
                        plsc.store_scatter(outb_b, [cj, rows16], e)
                return carry

            lax.fori_loop(0, C // L, group_body, 0)

        stage_idx(0, 0)
        gather(0, 0).start()
        stage_idx(1, 1)
        gather(1, 1).start()

        def chunk_body(g, carry):
            b = lax.rem(g, 2)

            def one(b):
                gather(g, b).wait()

                @pl.when(g >= 2)
                def _():
                    for cg in range(d // 8):
                        tile_write(g - 2, b, cg).wait()

                compute(g, b)
                for cg in range(d // 8):
                    tile_write(g, b, cg).start()

                @pl.when(g < n_chunks - 2)
                def _():
                    stage_idx(g + 2, b)
                    gather(g + 2, b).start()

            @pl.when(b == 0)
            def _():
                one(0)

            @pl.when(b == 1)
            def _():
                one(1)

            return carry

        lax.fori_loop(0, n_chunks, chunk_body, 0)
        for cg in range(d // 8):
            tile_write(n_chunks - 2, 0, cg).wait()
            tile_write(n_chunks - 1, 1, cg).wait()

    return k(x_flat, weight)


def kernel(x, weight):
    b, h = x.shape
    d = weight.shape[1]
    n = weight.shape[0]
    nfull = (n // C) * C
    tail_w = jnp.pad(weight[nfull:], ((0, C - (n - nfull)), (0, 0))).T
    w2 = _transpose_table(weight.T, tail_w)
    out5 = _softmax_embed(x.reshape(b * h), b, h, w2)
    return out5.transpose(0, 1, 3, 2, 4).reshape(h, d, b).transpose(2, 0, 1)
